# baseline (speedup 1.0000x reference)
import functools

import jax
import jax.numpy as jnp
from jax import lax
from jax.experimental import pallas as pl
from jax.experimental.pallas import tpu as pltpu
from jax.experimental.pallas import tpu_sc as plsc

VOCAB = 100000
EMBED = 64
B = 1024
L = 200
SCALE = 8.0

NC = 2
NS = 16
NW = NC * NS
EPW = B // NW
CH0 = 128
CH1 = L - CH0
NBUF = 4
RPI = 8

_mesh = plsc.VectorSubcoreMesh(core_axis_name="c", subcore_axis_name="s")


def _scale_buf(buf):
    @plsc.parallel_loop(0, L, step=RPI, unroll=2)
    def _(i):
        for r in range(RPI):
            for j in range(EMBED // 16):
                sl = pl.ds(j * 16, 16)
                buf[i + r, sl] = buf[i + r, sl] * SCALE


@functools.partial(
    pl.kernel,
    mesh=_mesh,
    out_type=jax.ShapeDtypeStruct((B, L, EMBED), jnp.float32),
    scratch_types=[
        pltpu.VMEM((EPW, L), jnp.int32),
        [pltpu.VMEM((L, EMBED), jnp.float32)] * NBUF,
        [pltpu.SemaphoreType.DMA] * NBUF,
        [pltpu.SemaphoreType.DMA] * NBUF,
    ],
    compiler_params=pltpu.CompilerParams(use_tc_tiling_on_sc=False),
)
def _embed_gather(idx_hbm, table_hbm, out_hbm, idx_v, bufs, gsems, osems):
    wid = lax.axis_index("s") * NC + lax.axis_index("c")
    e_base = wid * EPW
    pltpu.sync_copy(idx_hbm.at[pl.ds(e_base, EPW)], idx_v)

    ghandles = {}
    ohandles = {}

    for g in range(EPW + 1):
        b = g % NBUF
        if g < EPW:
            if g >= NBUF:
                ohandles[g - NBUF].wait()
            ghandles[g] = [
                pltpu.async_copy(
                    table_hbm.at[idx_v.at[g, pl.ds(0, CH0)]],
                    bufs[b].at[pl.ds(0, CH0)],
                    gsems[b]),
                pltpu.async_copy(
                    table_hbm.at[idx_v.at[g, pl.ds(CH0, CH1)]],
                    bufs[b].at[pl.ds(CH0, CH1)],
                    gsems[b]),
            ]
        if g >= 1:
            gp = g - 1
            bp = gp % NBUF
            for h in ghandles[gp]:
                h.wait()
            _scale_buf(bufs[bp])
            ohandles[gp] = pltpu.async_copy(
                bufs[bp], out_hbm.at[e_base + gp], osems[bp])

    for g in range(EPW - NBUF, EPW):
        ohandles[g].wait()


def kernel(tokens, input_embedding_table):
    return _embed_gather(tokens.astype(jnp.int32), input_embedding_table)
